# baseline (speedup 1.0000x reference)
import jax
import jax.numpy as jnp
import numpy as np
from jax.experimental import pallas as pl
from jax.experimental.pallas import tpu as pltpu
from jax.sharding import Mesh, PartitionSpec as P
from jax.experimental.shard_map import shard_map

_BN = 512


def _matmul_kernel(x_ref, w_ref, b_ref, o_ref):
    wb = w_ref[...].astype(jnp.bfloat16)
    acc = jax.lax.dot_general(
        x_ref[...], wb,
        dimension_numbers=(((1,), (1,)), ((), ())),
        preferred_element_type=jnp.float32,
    )
    o_ref[...] = acc + b_ref[...]


def _one_core(x, W, b):
    M, K = x.shape
    n = W.shape[0]
    return pl.pallas_call(
        _matmul_kernel,
        grid=(n // _BN,),
        in_specs=[
            pl.BlockSpec((M, K), lambda i: (0, 0)),
            pl.BlockSpec((_BN, K), lambda i: (i, 0)),
            pl.BlockSpec((1, _BN), lambda i: (0, i)),
        ],
        out_specs=pl.BlockSpec((M, _BN), lambda i: (0, i)),
        out_shape=jax.ShapeDtypeStruct((M, n), jnp.float32),
        compiler_params=pltpu.CompilerParams(
            dimension_semantics=("arbitrary",),
        ),
    )(x, W, b)


def kernel(x, W, b):
    M, K = x.shape
    N = W.shape[0]
    xb = x.astype(jnp.bfloat16)
    b2 = b.reshape(1, N)
    devs = jax.devices()
    mesh = Mesh(np.array(devs), ("d",))
    f = shard_map(
        _one_core,
        mesh=mesh,
        in_specs=(P(), P("d", None), P(None, "d")),
        out_specs=P(None, "d"),
        check_rep=False,
    )
    return f(xb, W, b2)


# speedup vs baseline: 18.5442x; 18.5442x over previous
import jax
import jax.numpy as jnp
from jax.experimental import pallas as pl
from jax.experimental.pallas import tpu as pltpu

_BN = 512
_KSPLIT = 4


def _matmul_kernel(x_ref, *refs):
    w_refs = refs[:_KSPLIT]
    b_ref = refs[_KSPLIT]
    o_ref = refs[_KSPLIT + 1]
    xb = x_ref[...].astype(jnp.bfloat16)
    kp = x_ref.shape[1] // _KSPLIT
    acc = None
    for j, w_ref in enumerate(w_refs):
        wb = w_ref[...].astype(jnp.bfloat16)
        part = jax.lax.dot_general(
            xb[:, j * kp:(j + 1) * kp], wb,
            dimension_numbers=(((1,), (1,)), ((), ())),
            preferred_element_type=jnp.float32,
        )
        acc = part if acc is None else acc + part
    o_ref[...] = acc + b_ref[...]


def kernel(x, W, b):
    M, K = x.shape
    N = W.shape[0]
    kp = K // _KSPLIT
    b2 = b.reshape(1, N)
    w_specs = [
        pl.BlockSpec((_BN, kp), lambda i, j=j: (i, j)) for j in range(_KSPLIT)
    ]
    out = pl.pallas_call(
        _matmul_kernel,
        grid=(N // _BN,),
        in_specs=[pl.BlockSpec((M, K), lambda i: (0, 0))]
        + w_specs
        + [pl.BlockSpec((1, _BN), lambda i: (0, i))],
        out_specs=pl.BlockSpec((M, _BN), lambda i: (0, i)),
        out_shape=jax.ShapeDtypeStruct((M, N), jnp.float32),
        compiler_params=pltpu.CompilerParams(
            dimension_semantics=("arbitrary",),
        ),
    )(x, *([W] * _KSPLIT), b2)
    return out
